# baseline (speedup 1.0000x reference)
import functools

import jax
import jax.numpy as jnp
from jax import lax
from jax.experimental import pallas as pl
from jax.experimental.pallas import tpu as pltpu
from jax.experimental.pallas import tpu_sc as plsc

N = 10000
NNZ = 320000
D = 128

NC = 2
NS = 16
NW = NC * NS
CHUNK = 64
NBUF = 5
CPW = 160
PER_W = CPW * CHUNK
PAD_TOTAL = NW * PER_W
NPAD = 10008
ROWS_PER_TILE = 632
LAST_ROWS = NPAD - 15 * ROWS_PER_TILE

_EPS = 1e-5


def _ln_mlp(xb, g0, b0, w1t, b1, g1, b1_, w2t, b2):
    m = jnp.mean(xb, axis=-1, keepdims=True)
    v = jnp.mean((xb - m) ** 2, axis=-1, keepdims=True)
    xn = (xb - m) * lax.rsqrt(v + _EPS) * g0 + b0
    hh = jnp.maximum(jnp.dot(xn, w1t, preferred_element_type=jnp.float32) + b1, 0.0)
    m2 = jnp.mean(hh, axis=-1, keepdims=True)
    v2 = jnp.mean((hh - m2) ** 2, axis=-1, keepdims=True)
    hn = (hh - m2) * lax.rsqrt(v2 + _EPS) * g1 + b1_
    return jnp.dot(hn, w2t, preferred_element_type=jnp.float32) + b2


def _enc_body(x_ref, g0, b0, w1t, b1, g1, b1_, w2t, b2, o_ref):
    o_ref[...] = jnp.maximum(
        _ln_mlp(x_ref[...], g0[...], b0[...], w1t[...], b1[...], g1[...],
                b1_[...], w2t[...], b2[...]), 0.0)


def _dec_body(p0_ref, p1_ref, g0, b0, w1t, b1, g1, b1_, w2t, b2, o_ref):
    agg = p0_ref[0] + p1_ref[0]
    o_ref[...] = jnp.maximum(
        _ln_mlp(agg, g0[...], b0[...], w1t[...], b1[...], g1[...],
                b1_[...], w2t[...], b2[...]), 0.0)


_ROWS_BLK = 5000
_GRID = N // _ROWS_BLK

def _wspecs():
    vec = lambda i: (0, 0)
    return [
        pl.BlockSpec((1, D), vec),
        pl.BlockSpec((1, D), vec),
        pl.BlockSpec((D, D), vec),
        pl.BlockSpec((1, D), vec),
        pl.BlockSpec((1, D), vec),
        pl.BlockSpec((1, D), vec),
        pl.BlockSpec((D, D), vec),
        pl.BlockSpec((1, D), vec),
    ]


def _enc_call(x, *w):
    return pl.pallas_call(
        _enc_body,
        grid=(_GRID,),
        in_specs=[pl.BlockSpec((_ROWS_BLK, D), lambda i: (i, 0))] + _wspecs(),
        out_specs=pl.BlockSpec((_ROWS_BLK, D), lambda i: (i, 0)),
        out_shape=jax.ShapeDtypeStruct((N, D), jnp.float32),
    )(x, *w)


def _dec_call(parts, *w):
    return pl.pallas_call(
        _dec_body,
        grid=(_GRID,),
        in_specs=[
            pl.BlockSpec((1, _ROWS_BLK, D), lambda i: (0, i, 0)),
            pl.BlockSpec((1, _ROWS_BLK, D), lambda i: (1, i, 0)),
        ] + _wspecs(),
        out_specs=pl.BlockSpec((_ROWS_BLK, D), lambda i: (i, 0)),
        out_shape=jax.ShapeDtypeStruct((N, D), jnp.float32),
    )(parts, parts, *w)


@functools.cache
def _sc_call():
    mesh = plsc.VectorSubcoreMesh(
        core_axis_name="c", subcore_axis_name="s",
        num_cores=NC, num_subcores=NS)
    return pl.kernel(
        _gather_segsum,
        out_type=jax.ShapeDtypeStruct((NC, NPAD, D), jnp.float32),
        mesh=mesh,
        scratch_types=[
            pltpu.VMEM((NBUF, CHUNK), jnp.int32),
            pltpu.VMEM((NBUF, CHUNK), jnp.int32),
            pltpu.VMEM((NBUF, CHUNK), jnp.int32),
            pltpu.VMEM((NBUF, CHUNK, D), jnp.float32),
            pltpu.VMEM_SHARED((NPAD, D), jnp.float32),
            pltpu.SemaphoreType.DMA((NBUF,)),
            pltpu.SemaphoreType.DMA((NBUF,)),
            pltpu.SemaphoreType.DMA((NBUF,)),
        ],
    )


def _gather_segsum(h_hbm, p_hbm, z_hbm, out_hbm,
                   pk, ibv, ibe, rows, acc, isem, gsem, ssem):
    c = lax.axis_index("c")
    s = lax.axis_index("s")
    wid = s * NC + c
    r0 = s * ROWS_PER_TILE
    base = wid * PER_W

    def _start_idx(j, b):
        pltpu.async_copy(p_hbm.at[pl.ds(base + j * CHUNK, CHUNK)],
                         pk.at[b], isem.at[b])

    def _wait_idx(j, b):
        pltpu.make_async_copy(p_hbm.at[pl.ds(base + j * CHUNK, CHUNK)],
                              pk.at[b], isem.at[b]).wait()

    def _unpack(b):
        for i in range(CHUNK // 16):
            p = pk[b, pl.ds(i * 16, 16)]
            ibv[b, pl.ds(i * 16, 16)] = p & 0xFFFF
            ibe[b, pl.ds(i * 16, 16)] = p >> 16

    def _start_gather(b):
        pltpu.async_copy(h_hbm.at[ibv.at[b]], rows.at[b], gsem.at[b])

    def _wait_gather(b):
        pltpu.make_async_copy(h_hbm.at[ibv.at[b]], rows.at[b],
                              gsem.at[b]).wait()

    def _start_scatter(b):
        pltpu.async_copy(rows.at[b], acc.at[ibe.at[b]], ssem.at[b], add=True)

    def _wait_scatter(b):
        pltpu.make_async_copy(rows.at[b], acc.at[ibe.at[b]],
                              ssem.at[b]).wait()

    @pl.when(s < NS - 1)
    def _():
        pltpu.sync_copy(z_hbm.at[pl.ds(r0, ROWS_PER_TILE)],
                        acc.at[pl.ds(r0, ROWS_PER_TILE)])

    @pl.when(s == NS - 1)
    def _():
        pltpu.sync_copy(z_hbm.at[pl.ds(r0, LAST_ROWS)],
                        acc.at[pl.ds(r0, LAST_ROWS)])

    plsc.subcore_barrier()

    _start_idx(0, 0)
    _start_idx(1, 1)
    _start_idx(2, 2)
    _wait_idx(0, 0)
    _unpack(0)
    _start_gather(0)
    _wait_idx(1, 1)
    _unpack(1)
    _start_gather(1)

    @pl.loop(0, CPW // NBUF)
    def _outer(jj):
        for b in range(NBUF):
            j = jj * NBUF + b
            b2 = (b + 2) % NBUF
            b3 = (b + 3) % NBUF

            @pl.when(j + 3 < CPW)
            def _():
                @pl.when(j >= NBUF - 3)
                def _():
                    _wait_scatter(b3)
                _start_idx(j + 3, b3)

            @pl.when(j + 2 < CPW)
            def _():
                _wait_idx(j + 2, b2)
                _unpack(b2)
                _start_gather(b2)

            _wait_gather(b)
            _start_scatter(b)

    for b in range(NBUF):
        _wait_scatter(b)
    plsc.subcore_barrier()

    @pl.when(s < NS - 1)
    def _():
        pltpu.sync_copy(acc.at[pl.ds(r0, ROWS_PER_TILE)],
                        out_hbm.at[c, pl.ds(r0, ROWS_PER_TILE)])

    @pl.when(s == NS - 1)
    def _():
        pltpu.sync_copy(acc.at[pl.ds(r0, LAST_ROWS)],
                        out_hbm.at[c, pl.ds(r0, LAST_ROWS)])


def kernel(x, V, E, enc_ln0_g, enc_ln0_b, enc_W1, enc_b1, enc_ln1_g,
           enc_ln1_b, enc_W2, enc_b2, dec_ln0_g, dec_ln0_b, dec_W1, dec_b1,
           dec_ln1_g, dec_ln1_b, dec_W2, dec_b2):
    pad = PAD_TOTAL - NNZ
    pad_v = jnp.arange(pad, dtype=jnp.int32) % N
    pad_e = N + jnp.arange(pad, dtype=jnp.int32) % (NPAD - N)
    Vp = jnp.concatenate([V, pad_v])
    Ep = jnp.concatenate([E, pad_e])
    packed = jnp.bitwise_or(Vp, Ep << 16)
    zeros = jnp.zeros((NPAD, D), jnp.float32)

    r = lambda a: a.reshape(1, D)
    enc_w = (r(enc_ln0_g), r(enc_ln0_b), enc_W1.T, r(enc_b1),
             r(enc_ln1_g), r(enc_ln1_b), enc_W2.T, r(enc_b2))
    dec_w = (r(dec_ln0_g), r(dec_ln0_b), dec_W1.T, r(dec_b1),
             r(dec_ln1_g), r(dec_ln1_b), dec_W2.T, r(dec_b2))

    h = _enc_call(x, *enc_w)
    parts = _sc_call()(h, packed, zeros)
    return _dec_call(parts, *dec_w)


# speedup vs baseline: 1.0231x; 1.0231x over previous
import functools

import jax
import jax.numpy as jnp
from jax import lax
from jax.experimental import pallas as pl
from jax.experimental.pallas import tpu as pltpu
from jax.experimental.pallas import tpu_sc as plsc

N = 10000
NNZ = 320000
D = 128

NC = 2
NS = 16
NW = NC * NS
CHUNK = 48
NBUF = 7
CPW = 217
PER_W = CPW * CHUNK
PAD_TOTAL = NW * PER_W
NPAD = 10008
ROWS_PER_TILE = 632
LAST_ROWS = NPAD - 15 * ROWS_PER_TILE

_EPS = 1e-5


def _ln_mlp(xb, g0, b0, w1t, b1, g1, b1_, w2t, b2):
    m = jnp.mean(xb, axis=-1, keepdims=True)
    v = jnp.mean((xb - m) ** 2, axis=-1, keepdims=True)
    xn = (xb - m) * lax.rsqrt(v + _EPS) * g0 + b0
    hh = jnp.maximum(jnp.dot(xn, w1t, preferred_element_type=jnp.float32) + b1, 0.0)
    m2 = jnp.mean(hh, axis=-1, keepdims=True)
    v2 = jnp.mean((hh - m2) ** 2, axis=-1, keepdims=True)
    hn = (hh - m2) * lax.rsqrt(v2 + _EPS) * g1 + b1_
    return jnp.dot(hn, w2t, preferred_element_type=jnp.float32) + b2


def _enc_body(x_ref, g0, b0, w1t, b1, g1, b1_, w2t, b2, o_ref):
    o_ref[...] = jnp.maximum(
        _ln_mlp(x_ref[...], g0[...], b0[...], w1t[...], b1[...], g1[...],
                b1_[...], w2t[...], b2[...]), 0.0)


def _dec_body(p0_ref, p1_ref, g0, b0, w1t, b1, g1, b1_, w2t, b2, o_ref):
    agg = p0_ref[0] + p1_ref[0]
    o_ref[...] = jnp.maximum(
        _ln_mlp(agg, g0[...], b0[...], w1t[...], b1[...], g1[...],
                b1_[...], w2t[...], b2[...]), 0.0)


_ROWS_BLK = 5000
_GRID = N // _ROWS_BLK

def _wspecs():
    vec = lambda i: (0, 0)
    return [
        pl.BlockSpec((1, D), vec),
        pl.BlockSpec((1, D), vec),
        pl.BlockSpec((D, D), vec),
        pl.BlockSpec((1, D), vec),
        pl.BlockSpec((1, D), vec),
        pl.BlockSpec((1, D), vec),
        pl.BlockSpec((D, D), vec),
        pl.BlockSpec((1, D), vec),
    ]


def _enc_call(x, *w):
    return pl.pallas_call(
        _enc_body,
        grid=(_GRID,),
        in_specs=[pl.BlockSpec((_ROWS_BLK, D), lambda i: (i, 0))] + _wspecs(),
        out_specs=pl.BlockSpec((_ROWS_BLK, D), lambda i: (i, 0)),
        out_shape=jax.ShapeDtypeStruct((N, D), jnp.float32),
    )(x, *w)


def _dec_call(parts, *w):
    return pl.pallas_call(
        _dec_body,
        grid=(_GRID,),
        in_specs=[
            pl.BlockSpec((1, _ROWS_BLK, D), lambda i: (0, i, 0)),
            pl.BlockSpec((1, _ROWS_BLK, D), lambda i: (1, i, 0)),
        ] + _wspecs(),
        out_specs=pl.BlockSpec((_ROWS_BLK, D), lambda i: (i, 0)),
        out_shape=jax.ShapeDtypeStruct((N, D), jnp.float32),
    )(parts, parts, *w)


@functools.cache
def _sc_call():
    mesh = plsc.VectorSubcoreMesh(
        core_axis_name="c", subcore_axis_name="s",
        num_cores=NC, num_subcores=NS)
    return pl.kernel(
        _gather_segsum,
        out_type=jax.ShapeDtypeStruct((NC, NPAD, D), jnp.float32),
        mesh=mesh,
        scratch_types=[
            pltpu.VMEM((NBUF, CHUNK), jnp.int32),
            pltpu.VMEM((NBUF, CHUNK), jnp.int32),
            pltpu.VMEM((NBUF, CHUNK), jnp.int32),
            pltpu.VMEM((NBUF, CHUNK, D), jnp.float32),
            pltpu.VMEM_SHARED((NPAD, D), jnp.float32),
            pltpu.SemaphoreType.DMA((NBUF,)),
            pltpu.SemaphoreType.DMA((NBUF,)),
            pltpu.SemaphoreType.DMA((NBUF,)),
        ],
    )


def _gather_segsum(h_hbm, p_hbm, z_hbm, out_hbm,
                   pk, ibv, ibe, rows, acc, isem, gsem, ssem):
    c = lax.axis_index("c")
    s = lax.axis_index("s")
    wid = s * NC + c
    r0 = s * ROWS_PER_TILE
    base = wid * PER_W

    def _start_idx(j, b):
        pltpu.async_copy(p_hbm.at[pl.ds(base + j * CHUNK, CHUNK)],
                         pk.at[b], isem.at[b])

    def _wait_idx(j, b):
        pltpu.make_async_copy(p_hbm.at[pl.ds(base + j * CHUNK, CHUNK)],
                              pk.at[b], isem.at[b]).wait()

    def _unpack(b):
        for i in range(CHUNK // 16):
            p = pk[b, pl.ds(i * 16, 16)]
            ibv[b, pl.ds(i * 16, 16)] = p & 0xFFFF
            ibe[b, pl.ds(i * 16, 16)] = p >> 16

    def _start_gather(b):
        pltpu.async_copy(h_hbm.at[ibv.at[b]], rows.at[b], gsem.at[b])

    def _wait_gather(b):
        pltpu.make_async_copy(h_hbm.at[ibv.at[b]], rows.at[b],
                              gsem.at[b]).wait()

    def _start_scatter(b):
        pltpu.async_copy(rows.at[b], acc.at[ibe.at[b]], ssem.at[b], add=True)

    def _wait_scatter(b):
        pltpu.make_async_copy(rows.at[b], acc.at[ibe.at[b]],
                              ssem.at[b]).wait()

    @pl.when(s < NS - 1)
    def _():
        pltpu.sync_copy(z_hbm.at[pl.ds(r0, ROWS_PER_TILE)],
                        acc.at[pl.ds(r0, ROWS_PER_TILE)])

    @pl.when(s == NS - 1)
    def _():
        pltpu.sync_copy(z_hbm.at[pl.ds(r0, LAST_ROWS)],
                        acc.at[pl.ds(r0, LAST_ROWS)])

    plsc.subcore_barrier()

    for t in range(4):
        _start_idx(t, t)
    for t in range(3):
        _wait_idx(t, t)
        _unpack(t)
        _start_gather(t)

    @pl.loop(0, CPW // NBUF)
    def _outer(jj):
        for b in range(NBUF):
            j = jj * NBUF + b
            b3 = (b + 3) % NBUF
            b4 = (b + 4) % NBUF

            @pl.when(j + 4 < CPW)
            def _():
                @pl.when(j >= NBUF - 4)
                def _():
                    _wait_scatter(b4)
                _start_idx(j + 4, b4)

            @pl.when(j + 3 < CPW)
            def _():
                _wait_idx(j + 3, b3)
                _unpack(b3)
                _start_gather(b3)

            _wait_gather(b)
            _start_scatter(b)

    for b in range(NBUF):
        _wait_scatter(b)
    plsc.subcore_barrier()

    @pl.when(s < NS - 1)
    def _():
        pltpu.sync_copy(acc.at[pl.ds(r0, ROWS_PER_TILE)],
                        out_hbm.at[c, pl.ds(r0, ROWS_PER_TILE)])

    @pl.when(s == NS - 1)
    def _():
        pltpu.sync_copy(acc.at[pl.ds(r0, LAST_ROWS)],
                        out_hbm.at[c, pl.ds(r0, LAST_ROWS)])


def kernel(x, V, E, enc_ln0_g, enc_ln0_b, enc_W1, enc_b1, enc_ln1_g,
           enc_ln1_b, enc_W2, enc_b2, dec_ln0_g, dec_ln0_b, dec_W1, dec_b1,
           dec_ln1_g, dec_ln1_b, dec_W2, dec_b2):
    pad = PAD_TOTAL - NNZ
    pad_v = jnp.arange(pad, dtype=jnp.int32) % N
    pad_e = N + jnp.arange(pad, dtype=jnp.int32) % (NPAD - N)
    Vp = jnp.concatenate([V, pad_v])
    Ep = jnp.concatenate([E, pad_e])
    packed = jnp.bitwise_or(Vp, Ep << 16)
    zeros = jnp.zeros((NPAD, D), jnp.float32)

    r = lambda a: a.reshape(1, D)
    enc_w = (r(enc_ln0_g), r(enc_ln0_b), enc_W1.T, r(enc_b1),
             r(enc_ln1_g), r(enc_ln1_b), enc_W2.T, r(enc_b2))
    dec_w = (r(dec_ln0_g), r(dec_ln0_b), dec_W1.T, r(dec_b1),
             r(dec_ln1_g), r(dec_ln1_b), dec_W2.T, r(dec_b2))

    h = _enc_call(x, *enc_w)
    parts = _sc_call()(h, packed, zeros)
    return _dec_call(parts, *dec_w)
